# baseline (speedup 1.0000x reference)
import functools

import jax
import jax.numpy as jnp
from jax import lax
from jax.experimental import pallas as pl
from jax.experimental.pallas import tpu as pltpu
from jax.experimental.pallas import tpu_sc as plsc

_L = 16

_GATHER_DNUMS = lax.GatherDimensionNumbers(
    offset_dims=(), collapsed_slice_dims=(0,), start_index_map=(0,))


def _lane_shuffle(x, idx):
    return lax.gather(x, idx[:, None], _GATHER_DNUMS, slice_sizes=(1,),
                      mode=lax.GatherScatterMode.PROMISE_IN_BOUNDS)


@functools.partial(
    pl.kernel,
    out_type=jax.ShapeDtypeStruct((225,), jnp.float32),
    mesh=plsc.VectorSubcoreMesh(core_axis_name="c", subcore_axis_name="s"),
    compiler_params=pltpu.CompilerParams(use_tc_tiling_on_sc=False),
    scratch_types=[
        pltpu.VMEM((_L,), jnp.int32),
        pltpu.VMEM((_L, 64), jnp.float32),
        pltpu.VMEM((_L, 32), jnp.float32),
        pltpu.VMEM((240,), jnp.float32),
        pltpu.SemaphoreType.DMA,
        pltpu.SemaphoreType.DMA,
    ],
)
def _sc_embed(v_hbm, sym_hbm, gram_hbm, out_hbm,
              v_v, sym_rows, gram_rows, out_v, sem_s, sem_g):
    wid = lax.axis_index("s") * 2 + lax.axis_index("c")

    @pl.when(wid == 0)
    def _():
        pltpu.sync_copy(v_hbm, v_v.at[pl.ds(0, 6)])
        vv = v_v[...]
        iota = lax.iota(jnp.int32, _L)
        sym_perm = jnp.where(iota < 3, iota, 0)
        sym_idx = _lane_shuffle(vv, sym_perm)
        gram_idx = _lane_shuffle(vv, jnp.full((_L,), 5, jnp.int32))
        cp_s = pltpu.async_copy(sym_hbm.at[sym_idx], sym_rows, sem_s)
        cp_g = pltpu.async_copy(gram_hbm.at[gram_idx], gram_rows, sem_g)
        cp_s.wait()
        cp_g.wait()
        for j in range(3):
            for c in range(4):
                out_v[pl.ds(j * 64 + c * _L, _L)] = sym_rows[j, pl.ds(c * _L, _L)]
        g_a = gram_rows[0, pl.ds(0, _L)]
        g_b = gram_rows[0, pl.ds(_L, _L)]
        v4f = _lane_shuffle(vv, jnp.full((_L,), 4, jnp.int32)).astype(jnp.float32)
        shr = jnp.maximum(iota - 1, 0)
        full15 = jnp.full((_L,), 15, jnp.int32)
        c1 = jnp.where(iota == 0, v4f, _lane_shuffle(g_a, shr))
        c2 = jnp.where(iota == 0, _lane_shuffle(g_a, full15),
                       _lane_shuffle(g_b, shr))
        c3 = _lane_shuffle(g_b, full15)
        out_v[pl.ds(192, _L)] = c1
        out_v[pl.ds(208, _L)] = c2
        out_v[pl.ds(224, _L)] = c3
        pltpu.sync_copy(out_v.at[pl.ds(0, 225)], out_hbm)


def kernel(v, symbol_table, grammar_table):
    return _sc_embed(v.astype(jnp.int32), symbol_table, grammar_table)


# speedup vs baseline: 1.6755x; 1.6755x over previous
import jax
import jax.numpy as jnp
from jax import lax
from jax.experimental import pallas as pl
from jax.experimental.pallas import tpu as pltpu
from jax.experimental.pallas import tpu_sc as plsc
from jax._src.pallas import mpmd

_L = 16

_SCS_MESH = plsc.ScalarSubcoreMesh(axis_name="c")
_TEC_MESH = plsc.VectorSubcoreMesh(core_axis_name="c", subcore_axis_name="s")


def _scs_fn(v_hbm, sym_hbm, gram_hbm, out_hbm,
            v_s, row0, row1, row2, grow, sem, gsem):
    del out_hbm

    @pl.when(lax.axis_index("c") == 0)
    def _():
        pltpu.sync_copy(v_hbm, v_s)
        cps = []
        for j, row in enumerate((row0, row1, row2)):
            cps.append(pltpu.async_copy(
                sym_hbm.at[pl.ds(v_s[j], 1), :], row, sem))
        cps.append(pltpu.async_copy(
            gram_hbm.at[pl.ds(v_s[5], 1), :], grow, sem))
        for cp in cps:
            cp.wait()
        pl.semaphore_signal(gsem, 1, device_id={"s": 0})


def _tec_fn(v_hbm, sym_hbm, gram_hbm, out_hbm,
            v_s, row0, row1, row2, grow, sem, gsem):
    del sym_hbm, gram_hbm, v_s, sem
    wid = lax.axis_index("s") * 2 + lax.axis_index("c")

    @pl.when(wid == 0)
    def _():
        pl.run_scoped(
            _tec_body(v_hbm, out_hbm, row0, row1, row2, grow, gsem),
            pltpu.VMEM((_L,), jnp.int32),
            pltpu.VMEM((32,), jnp.float32),
            pltpu.VMEM((240,), jnp.float32),
        )


def _tec_body(v_hbm, out_hbm, row0, row1, row2, grow, gsem):
    def body(v_v, g_v, out_v):
        pltpu.sync_copy(v_hbm, v_v.at[pl.ds(0, 6)])
        iota = lax.iota(jnp.int32, _L)
        v4f = plsc.load_gather(
            v_v, [jnp.full((_L,), 4, jnp.int32)]).astype(jnp.float32)
        pl.semaphore_wait(gsem, 1)
        for j, row in enumerate((row0, row1, row2)):
            pltpu.sync_copy(row.at[0], out_v.at[pl.ds(j * 64, 64)])
        pltpu.sync_copy(grow.at[0], g_v)
        c0 = jnp.where(
            iota == 0, v4f,
            plsc.load_gather(g_v, [jnp.maximum(iota - 1, 0)]))
        c1 = plsc.load_gather(g_v, [iota + 15])
        c2 = plsc.load_gather(g_v, [jnp.full((_L,), 31, jnp.int32)])
        out_v[pl.ds(192, _L)] = c0
        out_v[pl.ds(208, _L)] = c1
        out_v[pl.ds(224, _L)] = c2
        pltpu.sync_copy(out_v.at[pl.ds(0, 225)], out_hbm)
    return body


_sc_embed = mpmd.mpmd_map(
    [(_SCS_MESH, _scs_fn), (_TEC_MESH, _tec_fn)],
    out_types=jax.ShapeDtypeStruct((225,), jnp.float32),
    compiler_params=pltpu.CompilerParams(needs_layout_passes=False),
    scratch_types=[
        (pltpu.SMEM @ _SCS_MESH)((6,), jnp.int32),
        pltpu.VMEM_SHARED((1, 64), jnp.float32),
        pltpu.VMEM_SHARED((1, 64), jnp.float32),
        pltpu.VMEM_SHARED((1, 64), jnp.float32),
        pltpu.VMEM_SHARED((1, 32), jnp.float32),
        pltpu.SemaphoreType.DMA @ _SCS_MESH,
        pltpu.SemaphoreType.REGULAR @ _TEC_MESH,
    ],
)


def kernel(v, symbol_table, grammar_table):
    return _sc_embed(v.astype(jnp.int32), symbol_table, grammar_table)


# speedup vs baseline: 1.7542x; 1.0469x over previous
import functools

import jax
import jax.numpy as jnp
from jax.experimental import pallas as pl
from jax.experimental.pallas import tpu as pltpu


def _tc_body(v_ref, sym, gram, out_ref, r0, r1, r2, g, sems):
    cps = []
    for j, r in enumerate((r0, r1, r2)):
        cp = pltpu.make_async_copy(sym.at[pl.ds(v_ref[j], 1), :], r,
                                   sems.at[j])
        cp.start()
        cps.append(cp)
    cp_g = pltpu.make_async_copy(gram.at[pl.ds(v_ref[5], 1), :], g,
                                 sems.at[3])
    cp_g.start()
    for cp in cps:
        cp.wait()
    cp_g.wait()
    out_ref[pl.ds(0, 64)] = r0[0]
    out_ref[pl.ds(64, 64)] = r1[0]
    out_ref[pl.ds(128, 64)] = r2[0]
    out_ref[pl.ds(192, 1)] = v_ref[4].astype(jnp.float32)[None]
    out_ref[pl.ds(193, 32)] = g[0]


@functools.partial(jax.jit, static_argnames=())
def _tc_embed(v, symbol_table, grammar_table):
    grid_spec = pltpu.PrefetchScalarGridSpec(
        num_scalar_prefetch=1,
        grid=(1,),
        in_specs=[
            pl.BlockSpec(memory_space=pl.ANY),
            pl.BlockSpec(memory_space=pl.ANY),
        ],
        out_specs=pl.BlockSpec((225,), lambda i, v_ref: (0,)),
        scratch_shapes=[
            pltpu.VMEM((1, 64), jnp.float32),
            pltpu.VMEM((1, 64), jnp.float32),
            pltpu.VMEM((1, 64), jnp.float32),
            pltpu.VMEM((1, 32), jnp.float32),
            pltpu.SemaphoreType.DMA((4,)),
        ],
    )
    return pl.pallas_call(
        _tc_body,
        grid_spec=grid_spec,
        out_shape=jax.ShapeDtypeStruct((225,), jnp.float32),
    )(v, symbol_table, grammar_table)


def kernel(v, symbol_table, grammar_table):
    return _tc_embed(v.astype(jnp.int32), symbol_table, grammar_table)


# speedup vs baseline: 271.4470x; 154.7423x over previous
import functools

import jax
import jax.numpy as jnp
from jax.experimental import pallas as pl
from jax.experimental.pallas import tpu as pltpu


def _tc_body(v_ref, sym_t, gram_t, out_ref, r0, r1, r2, g, sems):
    cps = []
    for j, r in enumerate((r0, r1, r2)):
        base = (v_ref[j] // 128) * 128
        cp = pltpu.make_async_copy(sym_t.at[:, pl.ds(base, 128)], r,
                                   sems.at[j])
        cp.start()
        cps.append(cp)
    gbase = (v_ref[5] // 128) * 128
    cp_g = pltpu.make_async_copy(gram_t.at[:, pl.ds(gbase, 128)], g,
                                 sems.at[3])
    cp_g.start()
    lanes = jax.lax.broadcasted_iota(jnp.int32, (1, 128), 1)
    for cp in cps:
        cp.wait()
    cp_g.wait()
    for j, r in enumerate((r0, r1, r2)):
        onehot = (lanes == v_ref[j] % 128).astype(jnp.float32)
        out_ref[pl.ds(j * 64, 64)] = jnp.sum(r[...] * onehot, axis=1)
    g_onehot = (lanes == v_ref[5] % 128).astype(jnp.float32)
    out_ref[pl.ds(192, 1)] = v_ref[4].astype(jnp.float32)[None]
    out_ref[pl.ds(193, 32)] = jnp.sum(g[...] * g_onehot, axis=1)


def _tc_embed(v, sym_t, gram_t):
    grid_spec = pltpu.PrefetchScalarGridSpec(
        num_scalar_prefetch=1,
        grid=(1,),
        in_specs=[
            pl.BlockSpec(memory_space=pl.ANY),
            pl.BlockSpec(memory_space=pl.ANY),
        ],
        out_specs=pl.BlockSpec((225,), lambda i, v_ref: (0,)),
        scratch_shapes=[
            pltpu.VMEM((64, 128), jnp.float32),
            pltpu.VMEM((64, 128), jnp.float32),
            pltpu.VMEM((64, 128), jnp.float32),
            pltpu.VMEM((32, 128), jnp.float32),
            pltpu.SemaphoreType.DMA((4,)),
        ],
    )
    return pl.pallas_call(
        _tc_body,
        grid_spec=grid_spec,
        out_shape=jax.ShapeDtypeStruct((225,), jnp.float32),
    )(v, sym_t, gram_t)


def kernel(v, symbol_table, grammar_table):
    return _tc_embed(v.astype(jnp.int32), symbol_table.T, grammar_table.T)


# speedup vs baseline: 273.1246x; 1.0062x over previous
import functools

import jax
import jax.numpy as jnp
from jax.experimental import pallas as pl
from jax.experimental.pallas import tpu as pltpu


def _tc_body(v_ref, sym_t, gram_t, out_ref, r0, r1, r2, g, sems):
    cps = []
    for j, r in enumerate((r0, r1, r2)):
        base = (v_ref[j] // 128) * 128
        cp = pltpu.make_async_copy(sym_t.at[:, pl.ds(base, 128)], r,
                                   sems.at[j])
        cp.start()
        cps.append(cp)
    gbase = (v_ref[5] // 128) * 128
    cp_g = pltpu.make_async_copy(gram_t.at[:, pl.ds(gbase, 128)], g,
                                 sems.at[3])
    cp_g.start()
    lanes = jax.lax.broadcasted_iota(jnp.int32, (1, 128), 1)
    onehots = [(lanes == v_ref[j] % 128).astype(jnp.float32)
               for j in range(3)]
    g_onehot = (lanes == v_ref[5] % 128).astype(jnp.float32)
    out_ref[pl.ds(192, 1)] = v_ref[4].astype(jnp.float32)[None]
    for cp in cps:
        cp.wait()
    cp_g.wait()
    for j, r in enumerate((r0, r1, r2)):
        out_ref[pl.ds(j * 64, 64)] = jnp.sum(r[...] * onehots[j], axis=1)
    out_ref[pl.ds(193, 32)] = jnp.sum(g[...] * g_onehot, axis=1)


def _tc_embed(v, sym_t, gram_t):
    grid_spec = pltpu.PrefetchScalarGridSpec(
        num_scalar_prefetch=1,
        grid=(1,),
        in_specs=[
            pl.BlockSpec(memory_space=pl.ANY),
            pl.BlockSpec(memory_space=pl.ANY),
        ],
        out_specs=pl.BlockSpec((225,), lambda i, v_ref: (0,)),
        scratch_shapes=[
            pltpu.VMEM((64, 128), jnp.float32),
            pltpu.VMEM((64, 128), jnp.float32),
            pltpu.VMEM((64, 128), jnp.float32),
            pltpu.VMEM((32, 128), jnp.float32),
            pltpu.SemaphoreType.DMA((4,)),
        ],
    )
    return pl.pallas_call(
        _tc_body,
        grid_spec=grid_spec,
        out_shape=jax.ShapeDtypeStruct((225,), jnp.float32),
    )(v, sym_t, gram_t)


def kernel(v, symbol_table, grammar_table):
    return _tc_embed(v.astype(jnp.int32), symbol_table.T, grammar_table.T)
